# baseline (speedup 1.0000x reference)
import functools

import jax
import jax.numpy as jnp
from jax import lax
from jax.experimental import pallas as pl
from jax.experimental.pallas import tpu as pltpu
from jax.experimental.pallas import tpu_sc as plsc

F = 26
D = 64
B = 4096
V = 100000
NC, NS, L = 2, 16, 16
NW = NC * NS
BPW = B // NW
NCH = D // L
HALF = 13
B2 = 128
B2P = BPW // B2


def _sc_body(xT_hbm, w2_hbm, bias_hbm, emb_hbm, out_hbm, *scratch):
    idx_v, whi0, whi1, wr0, wr1, S_v, sqA_v, ffm_v, ret_v, bias_v = scratch[:10]
    row_bufs = scratch[10:10 + HALF]
    sem = scratch[10 + HALF]

    cid = lax.axis_index("c")
    sid = lax.axis_index("s")
    wid = sid * NC + cid
    base = wid * BPW

    pltpu.sync_copy(xT_hbm.at[:, pl.ds(base, BPW)], idx_v)
    pltpu.sync_copy(bias_hbm, bias_v)

    lane = lax.broadcasted_iota(jnp.int32, (L,), 0)
    _dnums = lax.GatherDimensionNumbers(
        offset_dims=(), collapsed_slice_dims=(0,), start_index_map=(0,))

    def _shuffle(v, perm):
        return lax.gather(
            v, perm[:, None], dimension_numbers=_dnums, slice_sizes=(1,),
            mode=lax.GatherScatterMode.PROMISE_IN_BOUNDS)

    def _lane_allsum(v):
        for sh in (1, 2, 4, 8):
            v = v + _shuffle(v, lane ^ sh)
        return v

    for bp in range(B2P):
        bb = bp * B2
        for half in range(2):
            f0 = half * HALF
            descs = [
                pltpu.async_copy(
                    emb_hbm.at[(f0 + j) // 2].at[idx_v.at[f0 + j, pl.ds(bb, B2)]],
                    row_bufs[j], sem)
                for j in range(HALF)
            ]
            for d_ in descs:
                d_.wait()

            def _chunks(j, b):
                off = ((f0 + j) & 1) * D
                regs = []
                for c2 in range(2):
                    ab = row_bufs[j][b, pl.ds(off + c2 * 2 * L, 2 * L)]
                    lo, hi = plsc.unpack(
                        ab, format=plsc.PackFormat.INTERLEAVED,
                        preferred_element_type=jnp.float32)
                    regs += [lo, hi]
                return regs

            if half == 0:
                def body_a(b, carry):
                    v = _chunks(0, b)
                    S = list(v)
                    q = [vv * vv for vv in v]
                    for j in range(1, HALF):
                        v = _chunks(j, b)
                        for c in range(NCH):
                            S[c] = S[c] + v[c]
                            q[c] = q[c] + v[c] * v[c]
                    for c in range(NCH):
                        S_v[b, pl.ds(c * L, L)] = S[c]
                    sqA_v[b, :] = (q[0] + q[1]) + (q[2] + q[3])
                    return carry

                lax.fori_loop(0, B2, body_a, 0)
            else:
                def body_b(b, vec):
                    S = [S_v[b, pl.ds(c * L, L)] for c in range(NCH)]
                    v = _chunks(0, b)
                    q = [vv * vv for vv in v]
                    for c in range(NCH):
                        S[c] = S[c] + v[c]
                    for j in range(1, HALF):
                        v = _chunks(j, b)
                        for c in range(NCH):
                            S[c] = S[c] + v[c]
                            q[c] = q[c] + v[c] * v[c]
                    p0 = S[0] * S[0] - q[0]
                    p1 = S[1] * S[1] - q[1]
                    p2 = S[2] * S[2] - q[2]
                    p3 = S[3] * S[3] - q[3]
                    s = _lane_allsum(((p0 + p1) + (p2 + p3)) - sqA_v[b, :])
                    bi = b & (L - 1)
                    vec = jnp.where(lane == bi, s, vec)

                    @pl.when(bi == L - 1)
                    def _():
                        ffm_v[pl.ds(bb + b - (L - 1), L)] = vec

                    return vec

                lax.fori_loop(0, B2, body_b, jnp.zeros((L,), jnp.float32))

    lane16 = lane
    bias_vec = bias_v[...]
    for g in range(BPW // L):
        sl = pl.ds(g * L, L)
        ret_v[sl] = bias_vec + 0.5 * ffm_v[sl]

    def fire_w(f, whi, wr):
        for g in range(BPW // L):
            sl = pl.ds(g * L, L)
            whi[sl] = lax.shift_right_logical(idx_v[f, sl], 3)
        return pltpu.async_copy(w2_hbm.at[f].at[whi], wr, sem)

    def reduce_w(f, wr):
        for g in range(BPW // L):
            sl = pl.ds(g * L, L)
            rows = lane16 + (g * L)
            cols = idx_v[f, sl] & 7
            vals = plsc.load_gather(wr, [rows, cols])
            plsc.addupdate(ret_v.at[sl], vals)

    slots = [(whi0, wr0), (whi1, wr1)]
    desc = fire_w(0, *slots[0])
    for f in range(F):
        nxt = None
        if f + 1 < F:
            nxt = fire_w(f + 1, *slots[(f + 1) % 2])
        desc.wait()
        reduce_w(f, slots[f % 2][1])
        desc = nxt

    pltpu.sync_copy(ret_v, out_hbm.at[pl.ds(base, BPW)])


_SC_SCRATCH = [
    pltpu.VMEM((F, BPW), jnp.int32),
    pltpu.VMEM((BPW,), jnp.int32),
    pltpu.VMEM((BPW,), jnp.int32),
    pltpu.VMEM((BPW, 8), jnp.float32),
    pltpu.VMEM((BPW, 8), jnp.float32),
    pltpu.VMEM((BPW, D), jnp.float32),
    pltpu.VMEM((BPW, L), jnp.float32),
    pltpu.VMEM((BPW,), jnp.float32),
    pltpu.VMEM((BPW,), jnp.float32),
    pltpu.VMEM((L,), jnp.float32),
] + [pltpu.VMEM((B2, 2 * D), jnp.bfloat16) for _ in range(HALF)] + [
    pltpu.SemaphoreType.DMA,
]

_sc_ffm = functools.partial(
    pl.kernel,
    out_type=jax.ShapeDtypeStruct((B,), jnp.float32),
    mesh=plsc.VectorSubcoreMesh(core_axis_name="c", subcore_axis_name="s"),
    scratch_types=_SC_SCRATCH,
    compiler_params=pltpu.CompilerParams(
        needs_layout_passes=False, use_tc_tiling_on_sc=False),
)(_sc_body)


_VB = 12800
_NJ = -(-V // _VB)
_WROWS = 20320
_WPAD = _WROWS * 128 - F * V


def _tc_body(emb_ref, w_ref, bias_ref, out_ref, reg_ref, acc_ref):
    i = pl.program_id(0)
    j = pl.program_id(1)
    xx = emb_ref[...].reshape(2 * D, _VB)
    out_ref[0] = jnp.transpose(xx).astype(jnp.bfloat16)

    @pl.when((i == 0) & (j == 0))
    def _():
        acc_ref[0] = 0.0

    r0 = lax.broadcasted_iota(jnp.int32, (2 * D, 2 * D), 0)
    r1 = lax.broadcasted_iota(jnp.int32, (2 * D, 2 * D), 1)
    eye = jnp.where(r0 == r1, 1.0, 0.0)
    dn = (((1,), (1,)), ((), ()))

    def _acc(xm):
        g = lax.dot_general(xm, xm, dn, preferred_element_type=jnp.float32)
        acc_ref[0] += jnp.sum(g * eye) * (1.0 / (V * D))

    @pl.when(j < _NJ - 1)
    def _():
        _acc(xx)

    @pl.when(j == _NJ - 1)
    def _():
        vpos = j * _VB + lax.broadcasted_iota(jnp.int32, (2 * D, _VB), 1)
        _acc(jnp.where(vpos < V, xx, 0.0))

    @pl.when((i == 0) & (j == 0))
    def _():
        w = w_ref[...]
        acc_ref[0] += jnp.sum(w * w) * (1.0 / (F * V)) + bias_ref[0] * bias_ref[0]

    @pl.when((i == F // 2 - 1) & (j == _NJ - 1))
    def _():
        reg_ref[0, 0] = acc_ref[0]


def _tc_pack_reg(emb3t, wpad, bias):
    return pl.pallas_call(
        _tc_body,
        grid=(F // 2, _NJ),
        in_specs=[
            pl.BlockSpec((2, D, _VB), lambda i, j: (i, 0, j)),
            pl.BlockSpec((_WROWS, 128), lambda i, j: (0, 0)),
            pl.BlockSpec(memory_space=pltpu.SMEM),
        ],
        out_specs=[
            pl.BlockSpec((1, _VB, 128), lambda i, j: (i, j, 0)),
            pl.BlockSpec(memory_space=pltpu.SMEM),
        ],
        out_shape=[
            jax.ShapeDtypeStruct((F // 2, V, 128), jnp.bfloat16),
            jax.ShapeDtypeStruct((1, 1), jnp.float32),
        ],
        scratch_shapes=[pltpu.SMEM((1,), jnp.float32)],
    )(emb3t, wpad, bias)


def kernel(x, W_lin, bias, emb):
    xT = x.T
    w2 = W_lin.reshape(F, V // 8, 8)
    bias16 = jnp.broadcast_to(bias, (L,))

    emb3t = jnp.transpose(emb, (0, 2, 1))
    wflat = W_lin.reshape(F * V)
    wpad = jnp.pad(wflat, (0, _WPAD)).reshape(_WROWS, 128)
    emb13, reg = _tc_pack_reg(emb3t, wpad, bias)

    ret_val = _sc_ffm(xT, w2, bias16, emb13)
    return (ret_val, reg[0, 0])


# speedup vs baseline: 2.6352x; 2.6352x over previous
import functools

import jax
import jax.numpy as jnp
from jax import lax
from jax.experimental import pallas as pl
from jax.experimental.pallas import tpu as pltpu
from jax.experimental.pallas import tpu_sc as plsc

F = 26
D = 64
B = 4096
V = 100000
NC, NS, L = 2, 16, 16
NW = NC * NS
BPW = B // NW
NCH = D // L
HALF = 13
B2 = 64
B2P = BPW // B2


def _sc_body(xT_hbm, w2_hbm, bias_hbm, emb_hbm, out_hbm, *scratch):
    idx_v, whi0, whi1, wr0, wr1, S_v, sqA_v, ffm_v, ret_v, bias_v = scratch[:10]
    row_bufs = scratch[10:10 + HALF]
    sem = scratch[10 + HALF]

    cid = lax.axis_index("c")
    sid = lax.axis_index("s")
    wid = sid * NC + cid
    base = wid * BPW

    pltpu.sync_copy(xT_hbm.at[:, pl.ds(base, BPW)], idx_v)
    pltpu.sync_copy(bias_hbm, bias_v)

    lane = lax.broadcasted_iota(jnp.int32, (L,), 0)
    _dnums = lax.GatherDimensionNumbers(
        offset_dims=(), collapsed_slice_dims=(0,), start_index_map=(0,))

    def _shuffle(v, perm):
        return lax.gather(
            v, perm[:, None], dimension_numbers=_dnums, slice_sizes=(1,),
            mode=lax.GatherScatterMode.PROMISE_IN_BOUNDS)

    def _lane_allsum(v):
        for sh in (1, 2, 4, 8):
            v = v + _shuffle(v, lane ^ sh)
        return v

    for bp in range(B2P):
        bb = bp * B2
        for half in range(2):
            f0 = half * HALF
            descs = [
                pltpu.async_copy(
                    emb_hbm.at[(f0 + j) // 2].at[idx_v.at[f0 + j, pl.ds(bb, B2)]],
                    row_bufs[j], sem)
                for j in range(HALF)
            ]
            for d_ in descs:
                d_.wait()

            def _chunks(j, b):
                off = ((f0 + j) & 1) * D
                return [row_bufs[j][b, pl.ds(off + c * L, L)] for c in range(NCH)]

            if half == 0:
                def body_a(b, carry):
                    v = _chunks(0, b)
                    S = list(v)
                    q = [vv * vv for vv in v]
                    for j in range(1, HALF):
                        v = _chunks(j, b)
                        for c in range(NCH):
                            S[c] = S[c] + v[c]
                            q[c] = q[c] + v[c] * v[c]
                    for c in range(NCH):
                        S_v[b, pl.ds(c * L, L)] = S[c]
                    sqA_v[b, :] = (q[0] + q[1]) + (q[2] + q[3])
                    return carry

                lax.fori_loop(0, B2, body_a, 0)
            else:
                def body_b(b, vec):
                    S = [S_v[b, pl.ds(c * L, L)] for c in range(NCH)]
                    v = _chunks(0, b)
                    q = [vv * vv for vv in v]
                    for c in range(NCH):
                        S[c] = S[c] + v[c]
                    for j in range(1, HALF):
                        v = _chunks(j, b)
                        for c in range(NCH):
                            S[c] = S[c] + v[c]
                            q[c] = q[c] + v[c] * v[c]
                    p0 = S[0] * S[0] - q[0]
                    p1 = S[1] * S[1] - q[1]
                    p2 = S[2] * S[2] - q[2]
                    p3 = S[3] * S[3] - q[3]
                    s = _lane_allsum(((p0 + p1) + (p2 + p3)) - sqA_v[b, :])
                    bi = b & (L - 1)
                    vec = jnp.where(lane == bi, s, vec)

                    @pl.when(bi == L - 1)
                    def _():
                        ffm_v[pl.ds(bb + b - (L - 1), L)] = vec

                    return vec

                lax.fori_loop(0, B2, body_b, jnp.zeros((L,), jnp.float32))

    lane16 = lane
    bias_vec = bias_v[...]
    for g in range(BPW // L):
        sl = pl.ds(g * L, L)
        ret_v[sl] = bias_vec + 0.5 * ffm_v[sl]

    def fire_w(f, whi, wr):
        for g in range(BPW // L):
            sl = pl.ds(g * L, L)
            whi[sl] = lax.shift_right_logical(idx_v[f, sl], 3)
        return pltpu.async_copy(w2_hbm.at[f].at[whi], wr, sem)

    def reduce_w(f, wr):
        for g in range(BPW // L):
            sl = pl.ds(g * L, L)
            rows = lane16 + (g * L)
            cols = idx_v[f, sl] & 7
            vals = plsc.load_gather(wr, [rows, cols])
            plsc.addupdate(ret_v.at[sl], vals)

    slots = [(whi0, wr0), (whi1, wr1)]
    desc = fire_w(0, *slots[0])
    for f in range(F):
        nxt = None
        if f + 1 < F:
            nxt = fire_w(f + 1, *slots[(f + 1) % 2])
        desc.wait()
        reduce_w(f, slots[f % 2][1])
        desc = nxt

    pltpu.sync_copy(ret_v, out_hbm.at[pl.ds(base, BPW)])


_SC_SCRATCH = [
    pltpu.VMEM((F, BPW), jnp.int32),
    pltpu.VMEM((BPW,), jnp.int32),
    pltpu.VMEM((BPW,), jnp.int32),
    pltpu.VMEM((BPW, 8), jnp.float32),
    pltpu.VMEM((BPW, 8), jnp.float32),
    pltpu.VMEM((B2, D), jnp.float32),
    pltpu.VMEM((B2, L), jnp.float32),
    pltpu.VMEM((BPW,), jnp.float32),
    pltpu.VMEM((BPW,), jnp.float32),
    pltpu.VMEM((L,), jnp.float32),
] + [pltpu.VMEM((B2, 2 * D), jnp.float32) for _ in range(HALF)] + [
    pltpu.SemaphoreType.DMA,
]

_sc_ffm = functools.partial(
    pl.kernel,
    out_type=jax.ShapeDtypeStruct((B,), jnp.float32),
    mesh=plsc.VectorSubcoreMesh(core_axis_name="c", subcore_axis_name="s"),
    scratch_types=_SC_SCRATCH,
    compiler_params=pltpu.CompilerParams(
        needs_layout_passes=False, use_tc_tiling_on_sc=False),
)(_sc_body)


_VB = 20480
_NJ = -(-V // _VB)
_NWB = 10
_WROWS = 20320
_WPAD = _WROWS * 128 - F * V


def _tc_body(emb_ref, w_ref, bias_ref, out_ref, reg_ref, acc_ref):
    i = pl.program_id(0)
    j = pl.program_id(1)
    k = i * _NJ + j
    xx = emb_ref[...].reshape(2 * D, _VB)
    out_ref[0] = jnp.transpose(xx)

    @pl.when(k == 0)
    def _():
        acc_ref[0] = bias_ref[0] * bias_ref[0]

    r0 = lax.broadcasted_iota(jnp.int32, (2 * D, 2 * D), 0)
    r1 = lax.broadcasted_iota(jnp.int32, (2 * D, 2 * D), 1)
    eye = jnp.where(r0 == r1, 1.0, 0.0)
    dn = (((1,), (1,)), ((), ()))

    def _acc(xm):
        g = lax.dot_general(xm, xm, dn, preferred_element_type=jnp.float32)
        acc_ref[0] += jnp.sum(g * eye) * (1.0 / (V * D))

    @pl.when(j < _NJ - 1)
    def _():
        _acc(xx)

    @pl.when(j == _NJ - 1)
    def _():
        vpos = j * _VB + lax.broadcasted_iota(jnp.int32, (2 * D, _VB), 1)
        _acc(jnp.where(vpos < V, xx, 0.0))

    @pl.when(k < _NWB)
    def _():
        w = w_ref[...]
        acc_ref[0] += jnp.sum(w * w) * (1.0 / (F * V))

    @pl.when((i == F // 2 - 1) & (j == _NJ - 1))
    def _():
        reg_ref[0, 0] = acc_ref[0]


def _tc_pack_reg(emb3t, wpad, bias):
    return pl.pallas_call(
        _tc_body,
        grid=(F // 2, _NJ),
        in_specs=[
            pl.BlockSpec((2, D, _VB), lambda i, j: (i, 0, j)),
            pl.BlockSpec((_WROWS // _NWB, 128),
                         lambda i, j: (jnp.minimum(i * _NJ + j, _NWB - 1), 0)),
            pl.BlockSpec(memory_space=pltpu.SMEM),
        ],
        out_specs=[
            pl.BlockSpec((1, _VB, 128), lambda i, j: (i, j, 0)),
            pl.BlockSpec(memory_space=pltpu.SMEM),
        ],
        out_shape=[
            jax.ShapeDtypeStruct((F // 2, _NJ * _VB, 128), jnp.float32),
            jax.ShapeDtypeStruct((1, 1), jnp.float32),
        ],
        scratch_shapes=[pltpu.SMEM((1,), jnp.float32)],
    )(emb3t, wpad, bias)


def kernel(x, W_lin, bias, emb):
    xT = x.T
    w2 = W_lin.reshape(F, V // 8, 8)
    bias16 = jnp.broadcast_to(bias, (L,))

    emb3t = jnp.transpose(emb, (0, 2, 1))
    wflat = W_lin.reshape(F * V)
    wpad = jnp.pad(wflat, (0, _WPAD)).reshape(_WROWS, 128)
    emb13, reg = _tc_pack_reg(emb3t, wpad, bias)

    ret_val = _sc_ffm(xT, w2, bias16, emb13)
    return (ret_val, reg[0, 0])


# speedup vs baseline: 2.7887x; 1.0582x over previous
import functools

import jax
import jax.numpy as jnp
from jax import lax
from jax.experimental import pallas as pl
from jax.experimental.pallas import tpu as pltpu
from jax.experimental.pallas import tpu_sc as plsc

F = 26
D = 64
B = 4096
V = 100000
NC, NS, L = 2, 16, 16
NW = NC * NS
BPW = B // NW
NCH = D // L
HALF = 13
B2 = 64
B2P = BPW // B2


def _sc_body(xT_hbm, w2_hbm, bias_hbm, emb_hbm, out_hbm, *scratch):
    (idx_v, gidx_v, par_v, whi0, whi1, wr0, wr1, S_v, sqA_v, ffm_v,
     ret_v, bias_v) = scratch[:12]
    row_bufs = scratch[12:12 + HALF]
    sem = scratch[12 + HALF]

    cid = lax.axis_index("c")
    sid = lax.axis_index("s")
    wid = sid * NC + cid
    base = wid * BPW

    pltpu.sync_copy(xT_hbm.at[:, pl.ds(base, BPW)], idx_v)
    pltpu.sync_copy(bias_hbm, bias_v)

    for f in range(F):
        for g in range(BPW // L):
            sl = pl.ds(g * L, L)
            w = idx_v[f, sl]
            gidx_v[f, sl] = (
                lax.shift_left(lax.shift_right_logical(w, 14), 13)
                | (w & (8192 - 1)))
            par_v[f, sl] = (lax.shift_right_logical(w, 13) & 1) * L

    lane = lax.broadcasted_iota(jnp.int32, (L,), 0)
    _dnums = lax.GatherDimensionNumbers(
        offset_dims=(), collapsed_slice_dims=(0,), start_index_map=(0,))

    def _shuffle(v, perm):
        return lax.gather(
            v, perm[:, None], dimension_numbers=_dnums, slice_sizes=(1,),
            mode=lax.GatherScatterMode.PROMISE_IN_BOUNDS)

    def _lane_allsum(v):
        for sh in (1, 2, 4, 8):
            v = v + _shuffle(v, lane ^ sh)
        return v

    for bp in range(B2P):
        bb = bp * B2
        for half in range(2):
            f0 = half * HALF
            descs = [
                pltpu.async_copy(
                    emb_hbm.at[(f0 + j) // 2].at[gidx_v.at[f0 + j, pl.ds(bb, B2)]],
                    row_bufs[j], sem)
                for j in range(HALF)
            ]
            for d_ in descs:
                d_.wait()

            def _chunks(j, b):
                off = ((f0 + j) & 1) * D
                band = b & (L - 1)
                pv = par_v[f0 + j, pl.ds(bb + (b - band), L)]
                sh = _shuffle(pv, jnp.full((L,), band, jnp.int32))
                sh = sh.astype(jnp.uint32)
                regs = []
                for c in range(NCH):
                    u = plsc.bitcast(row_bufs[j][b, pl.ds(off + c * L, L)],
                                     jnp.uint32)
                    bits = lax.shift_left(lax.shift_right_logical(u, sh),
                                          jnp.uint32(16))
                    regs.append(plsc.bitcast(bits, jnp.float32))
                return regs

            if half == 0:
                def body_a(b, carry):
                    v = _chunks(0, b)
                    S = list(v)
                    q = [vv * vv for vv in v]
                    for j in range(1, HALF):
                        v = _chunks(j, b)
                        for c in range(NCH):
                            S[c] = S[c] + v[c]
                            q[c] = q[c] + v[c] * v[c]
                    for c in range(NCH):
                        S_v[b, pl.ds(c * L, L)] = S[c]
                    sqA_v[b, :] = (q[0] + q[1]) + (q[2] + q[3])
                    return carry

                lax.fori_loop(0, B2, body_a, 0)
            else:
                def body_b(b, vec):
                    S = [S_v[b, pl.ds(c * L, L)] for c in range(NCH)]
                    v = _chunks(0, b)
                    q = [vv * vv for vv in v]
                    for c in range(NCH):
                        S[c] = S[c] + v[c]
                    for j in range(1, HALF):
                        v = _chunks(j, b)
                        for c in range(NCH):
                            S[c] = S[c] + v[c]
                            q[c] = q[c] + v[c] * v[c]
                    p0 = S[0] * S[0] - q[0]
                    p1 = S[1] * S[1] - q[1]
                    p2 = S[2] * S[2] - q[2]
                    p3 = S[3] * S[3] - q[3]
                    s = _lane_allsum(((p0 + p1) + (p2 + p3)) - sqA_v[b, :])
                    bi = b & (L - 1)
                    vec = jnp.where(lane == bi, s, vec)

                    @pl.when(bi == L - 1)
                    def _():
                        ffm_v[pl.ds(bb + b - (L - 1), L)] = vec

                    return vec

                lax.fori_loop(0, B2, body_b, jnp.zeros((L,), jnp.float32))

    lane16 = lane
    bias_vec = bias_v[...]
    for g in range(BPW // L):
        sl = pl.ds(g * L, L)
        ret_v[sl] = bias_vec + 0.5 * ffm_v[sl]

    def fire_w(f, whi, wr):
        for g in range(BPW // L):
            sl = pl.ds(g * L, L)
            whi[sl] = lax.shift_right_logical(idx_v[f, sl], 3)
        return pltpu.async_copy(w2_hbm.at[f].at[whi], wr, sem)

    def reduce_w(f, wr):
        for g in range(BPW // L):
            sl = pl.ds(g * L, L)
            rows = lane16 + (g * L)
            cols = idx_v[f, sl] & 7
            vals = plsc.load_gather(wr, [rows, cols])
            plsc.addupdate(ret_v.at[sl], vals)

    slots = [(whi0, wr0), (whi1, wr1)]
    desc = fire_w(0, *slots[0])
    for f in range(F):
        nxt = None
        if f + 1 < F:
            nxt = fire_w(f + 1, *slots[(f + 1) % 2])
        desc.wait()
        reduce_w(f, slots[f % 2][1])
        desc = nxt

    pltpu.sync_copy(ret_v, out_hbm.at[pl.ds(base, BPW)])


_SC_SCRATCH = [
    pltpu.VMEM((F, BPW), jnp.int32),
    pltpu.VMEM((F, BPW), jnp.int32),
    pltpu.VMEM((F, BPW), jnp.int32),
    pltpu.VMEM((BPW,), jnp.int32),
    pltpu.VMEM((BPW,), jnp.int32),
    pltpu.VMEM((BPW, 8), jnp.float32),
    pltpu.VMEM((BPW, 8), jnp.float32),
    pltpu.VMEM((B2, D), jnp.float32),
    pltpu.VMEM((B2, L), jnp.float32),
    pltpu.VMEM((BPW,), jnp.float32),
    pltpu.VMEM((BPW,), jnp.float32),
    pltpu.VMEM((L,), jnp.float32),
] + [pltpu.VMEM((B2, 2 * D), jnp.float32) for _ in range(HALF)] + [
    pltpu.SemaphoreType.DMA,
]

_sc_ffm = functools.partial(
    pl.kernel,
    out_type=jax.ShapeDtypeStruct((B,), jnp.float32),
    mesh=plsc.VectorSubcoreMesh(core_axis_name="c", subcore_axis_name="s"),
    scratch_types=_SC_SCRATCH,
    compiler_params=pltpu.CompilerParams(
        needs_layout_passes=False, use_tc_tiling_on_sc=False),
)(_sc_body)


_VB = 16384
_NJ = -(-V // _VB)
_NWB = 10
_WROWS = 20320
_WPAD = _WROWS * 128 - F * V


def _tc_body(emb_ref, w_ref, bias_ref, out_ref, reg_ref, acc_ref):
    i = pl.program_id(0)
    j = pl.program_id(1)
    k = i * _NJ + j
    xx = emb_ref[...].reshape(2 * D, _VB)
    a = xx[:, : _VB // 2]
    b = xx[:, _VB // 2 :]
    au = lax.bitcast_convert_type(a.astype(jnp.bfloat16), jnp.uint16).astype(jnp.uint32)
    bu = lax.bitcast_convert_type(b.astype(jnp.bfloat16), jnp.uint16).astype(jnp.uint32)
    w32 = lax.bitcast_convert_type(au | (bu << 16), jnp.float32)
    out_ref[0] = jnp.transpose(w32)

    @pl.when(k == 0)
    def _():
        acc_ref[0] = bias_ref[0] * bias_ref[0]

    r0 = lax.broadcasted_iota(jnp.int32, (2 * D, 2 * D), 0)
    r1 = lax.broadcasted_iota(jnp.int32, (2 * D, 2 * D), 1)
    eye = jnp.where(r0 == r1, 1.0, 0.0)
    dn = (((1,), (1,)), ((), ()))

    def _acc(xm):
        g = lax.dot_general(xm, xm, dn, preferred_element_type=jnp.float32)
        acc_ref[0] += jnp.sum(g * eye) * (1.0 / (V * D))

    @pl.when(j < _NJ - 1)
    def _():
        _acc(xx)

    @pl.when(j == _NJ - 1)
    def _():
        vpos = j * _VB + lax.broadcasted_iota(jnp.int32, (2 * D, _VB), 1)
        _acc(jnp.where(vpos < V, xx, 0.0))

    @pl.when(k < _NWB)
    def _():
        w = w_ref[...]
        acc_ref[0] += jnp.sum(w * w) * (1.0 / (F * V))

    @pl.when((i == F // 2 - 1) & (j == _NJ - 1))
    def _():
        reg_ref[0, 0] = acc_ref[0]


def _tc_pack_reg(emb3t, wpad, bias):
    return pl.pallas_call(
        _tc_body,
        grid=(F // 2, _NJ),
        in_specs=[
            pl.BlockSpec((2, D, _VB), lambda i, j: (i, 0, j)),
            pl.BlockSpec((_WROWS // _NWB, 128),
                         lambda i, j: (jnp.minimum(i * _NJ + j, _NWB - 1), 0)),
            pl.BlockSpec(memory_space=pltpu.SMEM),
        ],
        out_specs=[
            pl.BlockSpec((1, _VB // 2, 128), lambda i, j: (i, j, 0)),
            pl.BlockSpec(memory_space=pltpu.SMEM),
        ],
        out_shape=[
            jax.ShapeDtypeStruct((F // 2, _NJ * _VB // 2, 128), jnp.float32),
            jax.ShapeDtypeStruct((1, 1), jnp.float32),
        ],
        scratch_shapes=[pltpu.SMEM((1,), jnp.float32)],
    )(emb3t, wpad, bias)


def kernel(x, W_lin, bias, emb):
    xT = x.T
    w2 = W_lin.reshape(F, V // 8, 8)
    bias16 = jnp.broadcast_to(bias, (L,))

    emb3t = jnp.transpose(emb, (0, 2, 1))
    wflat = W_lin.reshape(F * V)
    wpad = jnp.pad(wflat, (0, _WPAD)).reshape(_WROWS, 128)
    emb13, reg = _tc_pack_reg(emb3t, wpad, bias)

    ret_val = _sc_ffm(xT, w2, bias16, emb13)
    return (ret_val, reg[0, 0])


# speedup vs baseline: 3.0315x; 1.0871x over previous
import functools

import jax
import jax.numpy as jnp
from jax import lax
from jax.experimental import pallas as pl
from jax.experimental.pallas import tpu as pltpu
from jax.experimental.pallas import tpu_sc as plsc

F = 26
D = 64
B = 4096
V = 100000
NC, NS, L = 2, 16, 16
NW = NC * NS
BPW = B // NW
NCH = D // L
HALF = 13
B2 = 64
B2P = BPW // B2


def _sc_body(xT_hbm, w2_hbm, bias_hbm, emb_hbm, out_hbm, *scratch):
    (idx_v, gidx_v, par_v, whi0, whi1, wr0, wr1, S_v, sqA_v, ffm_v,
     ret_v, bias_v) = scratch[:12]
    row_bufs = scratch[12:12 + HALF]
    sem = scratch[12 + HALF]

    cid = lax.axis_index("c")
    sid = lax.axis_index("s")
    wid = sid * NC + cid
    base = wid * BPW

    pltpu.sync_copy(xT_hbm.at[:, pl.ds(base, BPW)], idx_v)
    pltpu.sync_copy(bias_hbm, bias_v)

    VBH = 12544 // 2
    for f in range(F):
        for g in range(BPW // L):
            sl = pl.ds(g * L, L)
            w = idx_v[f, sl]
            jb = lax.shift_right_logical(
                lax.shift_right_logical(w, 8) * 1338, 16)
            r = w - jb * 12544
            hi = r >= VBH
            gidx_v[f, sl] = jb * VBH + jnp.where(hi, r - VBH, r)
            par_v[f, sl] = jnp.where(hi, L, 0)

    lane = lax.broadcasted_iota(jnp.int32, (L,), 0)
    _dnums = lax.GatherDimensionNumbers(
        offset_dims=(), collapsed_slice_dims=(0,), start_index_map=(0,))

    def _shuffle(v, perm):
        return lax.gather(
            v, perm[:, None], dimension_numbers=_dnums, slice_sizes=(1,),
            mode=lax.GatherScatterMode.PROMISE_IN_BOUNDS)

    def _lane_allsum(v):
        for sh in (1, 2, 4, 8):
            v = v + _shuffle(v, lane ^ sh)
        return v

    for bp in range(B2P):
        bb = bp * B2
        for half in range(2):
            f0 = half * HALF
            descs = [
                pltpu.async_copy(
                    emb_hbm.at[(f0 + j) // 2].at[gidx_v.at[f0 + j, pl.ds(bb, B2)]],
                    row_bufs[j], sem)
                for j in range(HALF)
            ]
            for d_ in descs:
                d_.wait()

            def _chunks(j, b):
                off = ((f0 + j) & 1) * D
                band = b & (L - 1)
                pv = par_v[f0 + j, pl.ds(bb + (b - band), L)]
                sh = _shuffle(pv, jnp.full((L,), band, jnp.int32))
                sh = sh.astype(jnp.uint32)
                regs = []
                for c in range(NCH):
                    u = plsc.bitcast(row_bufs[j][b, pl.ds(off + c * L, L)],
                                     jnp.uint32)
                    bits = lax.shift_left(lax.shift_right_logical(u, sh),
                                          jnp.uint32(16))
                    regs.append(plsc.bitcast(bits, jnp.float32))
                return regs

            if half == 0:
                def body_a(b, carry):
                    v = _chunks(0, b)
                    S = list(v)
                    q = [vv * vv for vv in v]
                    for j in range(1, HALF):
                        v = _chunks(j, b)
                        for c in range(NCH):
                            S[c] = S[c] + v[c]
                            q[c] = q[c] + v[c] * v[c]
                    for c in range(NCH):
                        S_v[b, pl.ds(c * L, L)] = S[c]
                    sqA_v[b, :] = (q[0] + q[1]) + (q[2] + q[3])
                    return carry

                lax.fori_loop(0, B2, body_a, 0)
            else:
                def body_b(b, vec):
                    S = [S_v[b, pl.ds(c * L, L)] for c in range(NCH)]
                    v = _chunks(0, b)
                    q = [vv * vv for vv in v]
                    for c in range(NCH):
                        S[c] = S[c] + v[c]
                    for j in range(1, HALF):
                        v = _chunks(j, b)
                        for c in range(NCH):
                            S[c] = S[c] + v[c]
                            q[c] = q[c] + v[c] * v[c]
                    p0 = S[0] * S[0] - q[0]
                    p1 = S[1] * S[1] - q[1]
                    p2 = S[2] * S[2] - q[2]
                    p3 = S[3] * S[3] - q[3]
                    s = _lane_allsum(((p0 + p1) + (p2 + p3)) - sqA_v[b, :])
                    bi = b & (L - 1)
                    vec = jnp.where(lane == bi, s, vec)

                    @pl.when(bi == L - 1)
                    def _():
                        ffm_v[pl.ds(bb + b - (L - 1), L)] = vec

                    return vec

                lax.fori_loop(0, B2, body_b, jnp.zeros((L,), jnp.float32))

    lane16 = lane
    bias_vec = bias_v[...]
    for g in range(BPW // L):
        sl = pl.ds(g * L, L)
        ret_v[sl] = bias_vec + 0.5 * ffm_v[sl]

    def fire_w(f, whi, wr):
        for g in range(BPW // L):
            sl = pl.ds(g * L, L)
            whi[sl] = lax.shift_right_logical(idx_v[f, sl], 3)
        return pltpu.async_copy(w2_hbm.at[f].at[whi], wr, sem)

    def reduce_w(f, wr):
        for g in range(BPW // L):
            sl = pl.ds(g * L, L)
            rows = lane16 + (g * L)
            cols = idx_v[f, sl] & 7
            vals = plsc.load_gather(wr, [rows, cols])
            plsc.addupdate(ret_v.at[sl], vals)

    slots = [(whi0, wr0), (whi1, wr1)]
    desc = fire_w(0, *slots[0])
    for f in range(F):
        nxt = None
        if f + 1 < F:
            nxt = fire_w(f + 1, *slots[(f + 1) % 2])
        desc.wait()
        reduce_w(f, slots[f % 2][1])
        desc = nxt

    pltpu.sync_copy(ret_v, out_hbm.at[pl.ds(base, BPW)])


_SC_SCRATCH = [
    pltpu.VMEM((F, BPW), jnp.int32),
    pltpu.VMEM((F, BPW), jnp.int32),
    pltpu.VMEM((F, BPW), jnp.int32),
    pltpu.VMEM((BPW,), jnp.int32),
    pltpu.VMEM((BPW,), jnp.int32),
    pltpu.VMEM((BPW, 8), jnp.float32),
    pltpu.VMEM((BPW, 8), jnp.float32),
    pltpu.VMEM((B2, D), jnp.float32),
    pltpu.VMEM((B2, L), jnp.float32),
    pltpu.VMEM((BPW,), jnp.float32),
    pltpu.VMEM((BPW,), jnp.float32),
    pltpu.VMEM((L,), jnp.float32),
] + [pltpu.VMEM((B2, 2 * D), jnp.float32) for _ in range(HALF)] + [
    pltpu.SemaphoreType.DMA,
]

_sc_ffm = functools.partial(
    pl.kernel,
    out_type=jax.ShapeDtypeStruct((B,), jnp.float32),
    mesh=plsc.VectorSubcoreMesh(core_axis_name="c", subcore_axis_name="s"),
    scratch_types=_SC_SCRATCH,
    compiler_params=pltpu.CompilerParams(
        needs_layout_passes=False, use_tc_tiling_on_sc=False),
)(_sc_body)


_VB = 12544
_NJ = -(-V // _VB)
_NWB = 10
_WROWS = 20320
_WPAD = _WROWS * 128 - F * V


def _tc_body(emb_ref, w_ref, bias_ref, out_ref, reg_ref, acc_ref):
    i = pl.program_id(0)
    j = pl.program_id(1)
    k = i * _NJ + j
    xx = emb_ref[...].reshape(2 * D, _VB)
    a = xx[:, : _VB // 2]
    b = xx[:, _VB // 2 :]
    au = lax.bitcast_convert_type(a.astype(jnp.bfloat16), jnp.uint16).astype(jnp.uint32)
    bu = lax.bitcast_convert_type(b.astype(jnp.bfloat16), jnp.uint16).astype(jnp.uint32)
    w32 = lax.bitcast_convert_type(au | (bu << 16), jnp.float32)
    out_ref[0] = jnp.transpose(w32)

    @pl.when(k == 0)
    def _():
        acc_ref[0] = bias_ref[0] * bias_ref[0]

    r0 = lax.broadcasted_iota(jnp.int32, (2 * D, 2 * D), 0)
    r1 = lax.broadcasted_iota(jnp.int32, (2 * D, 2 * D), 1)
    eye = jnp.where(r0 == r1, 1.0, 0.0)
    dn = (((1,), (1,)), ((), ()))

    def _acc(xm):
        g = lax.dot_general(xm, xm, dn, preferred_element_type=jnp.float32)
        acc_ref[0] += jnp.sum(g * eye) * (1.0 / (V * D))

    @pl.when(j < _NJ - 1)
    def _():
        _acc(xx)

    @pl.when(j == _NJ - 1)
    def _():
        vpos = j * _VB + lax.broadcasted_iota(jnp.int32, (2 * D, _VB), 1)
        _acc(jnp.where(vpos < V, xx, 0.0))

    @pl.when(k < _NWB)
    def _():
        w = w_ref[...]
        acc_ref[0] += jnp.sum(w * w) * (1.0 / (F * V))

    @pl.when((i == F // 2 - 1) & (j == _NJ - 1))
    def _():
        reg_ref[0, 0] = acc_ref[0]


def _tc_pack_reg(emb3t, wpad, bias):
    return pl.pallas_call(
        _tc_body,
        grid=(F // 2, _NJ),
        in_specs=[
            pl.BlockSpec((2, D, _VB), lambda i, j: (i, 0, j)),
            pl.BlockSpec((_WROWS // _NWB, 128),
                         lambda i, j: (jnp.minimum(i * _NJ + j, _NWB - 1), 0)),
            pl.BlockSpec(memory_space=pltpu.SMEM),
        ],
        out_specs=[
            pl.BlockSpec((1, _VB // 2, 128), lambda i, j: (i, j, 0)),
            pl.BlockSpec(memory_space=pltpu.SMEM),
        ],
        out_shape=[
            jax.ShapeDtypeStruct((F // 2, _NJ * _VB // 2, 128), jnp.float32),
            jax.ShapeDtypeStruct((1, 1), jnp.float32),
        ],
        scratch_shapes=[pltpu.SMEM((1,), jnp.float32)],
    )(emb3t, wpad, bias)


def kernel(x, W_lin, bias, emb):
    xT = x.T
    w2 = W_lin.reshape(F, V // 8, 8)
    bias16 = jnp.broadcast_to(bias, (L,))

    emb3t = jnp.transpose(emb, (0, 2, 1))
    wflat = W_lin.reshape(F * V)
    wpad = jnp.pad(wflat, (0, _WPAD)).reshape(_WROWS, 128)
    emb13, reg = _tc_pack_reg(emb3t, wpad, bias)

    ret_val = _sc_ffm(xT, w2, bias16, emb13)
    return (ret_val, reg[0, 0])


# speedup vs baseline: 3.0331x; 1.0005x over previous
import functools

import jax
import jax.numpy as jnp
from jax import lax
from jax.experimental import pallas as pl
from jax.experimental.pallas import tpu as pltpu
from jax.experimental.pallas import tpu_sc as plsc

F = 26
D = 64
B = 4096
V = 100000
NC, NS, L = 2, 16, 16
NW = NC * NS
BPW = B // NW
NCH = D // L
HALF = 13
B2 = 64
B2P = BPW // B2


def _sc_body(xT_hbm, w2_hbm, bias_hbm, emb_hbm, out_hbm, *scratch):
    (idx_v, gidx_v, par_v, whi0, whi1, wr0, wr1, S_v, sqA_v, ffm_v,
     ret_v, bias_v) = scratch[:12]
    row_bufs = scratch[12:12 + HALF]
    sem = scratch[12 + HALF]

    cid = lax.axis_index("c")
    sid = lax.axis_index("s")
    wid = sid * NC + cid
    base = wid * BPW

    pltpu.sync_copy(xT_hbm.at[:, pl.ds(base, BPW)], idx_v)
    pltpu.sync_copy(bias_hbm, bias_v)

    VBH = 25088 // 2
    for f in range(F):
        for g in range(BPW // L):
            sl = pl.ds(g * L, L)
            w = idx_v[f, sl]
            jb = lax.shift_right_logical(
                lax.shift_right_logical(w, 9) * 1338, 16)
            r = w - jb * 25088
            hi = r >= VBH
            gidx_v[f, sl] = jb * VBH + jnp.where(hi, r - VBH, r)
            par_v[f, sl] = jnp.where(hi, L, 0)

    lane = lax.broadcasted_iota(jnp.int32, (L,), 0)
    _dnums = lax.GatherDimensionNumbers(
        offset_dims=(), collapsed_slice_dims=(0,), start_index_map=(0,))

    def _shuffle(v, perm):
        return lax.gather(
            v, perm[:, None], dimension_numbers=_dnums, slice_sizes=(1,),
            mode=lax.GatherScatterMode.PROMISE_IN_BOUNDS)

    def _lane_allsum(v):
        for sh in (1, 2, 4, 8):
            v = v + _shuffle(v, lane ^ sh)
        return v

    for bp in range(B2P):
        bb = bp * B2
        for half in range(2):
            f0 = half * HALF
            descs = [
                pltpu.async_copy(
                    emb_hbm.at[(f0 + j) // 2].at[gidx_v.at[f0 + j, pl.ds(bb, B2)]],
                    row_bufs[j], sem)
                for j in range(HALF)
            ]
            for d_ in descs:
                d_.wait()

            def _chunks(j, b):
                off = ((f0 + j) & 1) * D
                band = b & (L - 1)
                pv = par_v[f0 + j, pl.ds(bb + (b - band), L)]
                sh = _shuffle(pv, jnp.full((L,), band, jnp.int32))
                sh = sh.astype(jnp.uint32)
                regs = []
                for c in range(NCH):
                    u = plsc.bitcast(row_bufs[j][b, pl.ds(off + c * L, L)],
                                     jnp.uint32)
                    bits = lax.shift_left(lax.shift_right_logical(u, sh),
                                          jnp.uint32(16))
                    regs.append(plsc.bitcast(bits, jnp.float32))
                return regs

            if half == 0:
                def body_a(b, carry):
                    v = _chunks(0, b)
                    S = list(v)
                    q = [vv * vv for vv in v]
                    for j in range(1, HALF):
                        v = _chunks(j, b)
                        for c in range(NCH):
                            S[c] = S[c] + v[c]
                            q[c] = q[c] + v[c] * v[c]
                    for c in range(NCH):
                        S_v[b, pl.ds(c * L, L)] = S[c]
                    sqA_v[b, :] = (q[0] + q[1]) + (q[2] + q[3])
                    return carry

                lax.fori_loop(0, B2, body_a, 0)
            else:
                def body_b(b, vec):
                    S = [S_v[b, pl.ds(c * L, L)] for c in range(NCH)]
                    v = _chunks(0, b)
                    q = [vv * vv for vv in v]
                    for c in range(NCH):
                        S[c] = S[c] + v[c]
                    for j in range(1, HALF):
                        v = _chunks(j, b)
                        for c in range(NCH):
                            S[c] = S[c] + v[c]
                            q[c] = q[c] + v[c] * v[c]
                    p0 = S[0] * S[0] - q[0]
                    p1 = S[1] * S[1] - q[1]
                    p2 = S[2] * S[2] - q[2]
                    p3 = S[3] * S[3] - q[3]
                    s = _lane_allsum(((p0 + p1) + (p2 + p3)) - sqA_v[b, :])
                    bi = b & (L - 1)
                    vec = jnp.where(lane == bi, s, vec)

                    @pl.when(bi == L - 1)
                    def _():
                        ffm_v[pl.ds(bb + b - (L - 1), L)] = vec

                    return vec

                lax.fori_loop(0, B2, body_b, jnp.zeros((L,), jnp.float32))

    lane16 = lane
    bias_vec = bias_v[...]
    for g in range(BPW // L):
        sl = pl.ds(g * L, L)
        ret_v[sl] = bias_vec + 0.5 * ffm_v[sl]

    def fire_w(f, whi, wr):
        for g in range(BPW // L):
            sl = pl.ds(g * L, L)
            whi[sl] = lax.shift_right_logical(idx_v[f, sl], 3)
        return pltpu.async_copy(w2_hbm.at[f].at[whi], wr, sem)

    def reduce_w(f, wr):
        for g in range(BPW // L):
            sl = pl.ds(g * L, L)
            rows = lane16 + (g * L)
            cols = idx_v[f, sl] & 7
            vals = plsc.load_gather(wr, [rows, cols])
            plsc.addupdate(ret_v.at[sl], vals)

    slots = [(whi0, wr0), (whi1, wr1)]
    desc = fire_w(0, *slots[0])
    for f in range(F):
        nxt = None
        if f + 1 < F:
            nxt = fire_w(f + 1, *slots[(f + 1) % 2])
        desc.wait()
        reduce_w(f, slots[f % 2][1])
        desc = nxt

    pltpu.sync_copy(ret_v, out_hbm.at[pl.ds(base, BPW)])


_SC_SCRATCH = [
    pltpu.VMEM((F, BPW), jnp.int32),
    pltpu.VMEM((F, BPW), jnp.int32),
    pltpu.VMEM((F, BPW), jnp.int32),
    pltpu.VMEM((BPW,), jnp.int32),
    pltpu.VMEM((BPW,), jnp.int32),
    pltpu.VMEM((BPW, 8), jnp.float32),
    pltpu.VMEM((BPW, 8), jnp.float32),
    pltpu.VMEM((B2, D), jnp.float32),
    pltpu.VMEM((B2, L), jnp.float32),
    pltpu.VMEM((BPW,), jnp.float32),
    pltpu.VMEM((BPW,), jnp.float32),
    pltpu.VMEM((L,), jnp.float32),
] + [pltpu.VMEM((B2, 2 * D), jnp.float32) for _ in range(HALF)] + [
    pltpu.SemaphoreType.DMA,
]

_sc_ffm = functools.partial(
    pl.kernel,
    out_type=jax.ShapeDtypeStruct((B,), jnp.float32),
    mesh=plsc.VectorSubcoreMesh(core_axis_name="c", subcore_axis_name="s"),
    scratch_types=_SC_SCRATCH,
    compiler_params=pltpu.CompilerParams(
        needs_layout_passes=False, use_tc_tiling_on_sc=False),
)(_sc_body)


_VB = 25088
_NJ = -(-V // _VB)
_NWB = 10
_WROWS = 20320
_WPAD = _WROWS * 128 - F * V


def _tc_body(emb_ref, w_ref, bias_ref, out_ref, reg_ref, acc_ref):
    i = pl.program_id(0)
    j = pl.program_id(1)
    k = i * _NJ + j
    xx = emb_ref[...].reshape(2 * D, _VB)
    a = xx[:, : _VB // 2]
    b = xx[:, _VB // 2 :]
    au = lax.bitcast_convert_type(a.astype(jnp.bfloat16), jnp.uint16).astype(jnp.uint32)
    bu = lax.bitcast_convert_type(b.astype(jnp.bfloat16), jnp.uint16).astype(jnp.uint32)
    w32 = lax.bitcast_convert_type(au | (bu << 16), jnp.float32)
    out_ref[0] = jnp.transpose(w32)

    @pl.when(k == 0)
    def _():
        acc_ref[0] = bias_ref[0] * bias_ref[0]

    r0 = lax.broadcasted_iota(jnp.int32, (2 * D, 2 * D), 0)
    r1 = lax.broadcasted_iota(jnp.int32, (2 * D, 2 * D), 1)
    eye = jnp.where(r0 == r1, 1.0, 0.0)
    dn = (((1,), (1,)), ((), ()))

    def _acc(xm):
        g = lax.dot_general(xm, xm, dn, preferred_element_type=jnp.float32)
        acc_ref[0] += jnp.sum(g * eye) * (1.0 / (V * D))

    @pl.when(j < _NJ - 1)
    def _():
        _acc(xx)

    @pl.when(j == _NJ - 1)
    def _():
        vpos = j * _VB + lax.broadcasted_iota(jnp.int32, (2 * D, _VB), 1)
        _acc(jnp.where(vpos < V, xx, 0.0))

    @pl.when(k < _NWB)
    def _():
        w = w_ref[...]
        acc_ref[0] += jnp.sum(w * w) * (1.0 / (F * V))

    @pl.when((i == F // 2 - 1) & (j == _NJ - 1))
    def _():
        reg_ref[0, 0] = acc_ref[0]


def _tc_pack_reg(emb3t, wpad, bias):
    return pl.pallas_call(
        _tc_body,
        grid=(F // 2, _NJ),
        in_specs=[
            pl.BlockSpec((2, D, _VB), lambda i, j: (i, 0, j)),
            pl.BlockSpec((_WROWS // _NWB, 128),
                         lambda i, j: (jnp.minimum(i * _NJ + j, _NWB - 1), 0)),
            pl.BlockSpec(memory_space=pltpu.SMEM),
        ],
        out_specs=[
            pl.BlockSpec((1, _VB // 2, 128), lambda i, j: (i, j, 0)),
            pl.BlockSpec(memory_space=pltpu.SMEM),
        ],
        out_shape=[
            jax.ShapeDtypeStruct((F // 2, _NJ * _VB // 2, 128), jnp.float32),
            jax.ShapeDtypeStruct((1, 1), jnp.float32),
        ],
        scratch_shapes=[pltpu.SMEM((1,), jnp.float32)],
    )(emb3t, wpad, bias)


def kernel(x, W_lin, bias, emb):
    xT = x.T
    w2 = W_lin.reshape(F, V // 8, 8)
    bias16 = jnp.broadcast_to(bias, (L,))

    emb3t = jnp.transpose(emb, (0, 2, 1))
    wflat = W_lin.reshape(F * V)
    wpad = jnp.pad(wflat, (0, _WPAD)).reshape(_WROWS, 128)
    emb13, reg = _tc_pack_reg(emb3t, wpad, bias)

    ret_val = _sc_ffm(xT, w2, bias16, emb13)
    return (ret_val, reg[0, 0])


# speedup vs baseline: 3.0564x; 1.0077x over previous
import functools

import jax
import jax.numpy as jnp
from jax import lax
from jax.experimental import pallas as pl
from jax.experimental.pallas import tpu as pltpu
from jax.experimental.pallas import tpu_sc as plsc

F = 26
D = 64
B = 4096
V = 100000
NC, NS, L = 2, 16, 16
NW = NC * NS
BPW = B // NW
NCH = D // L
HALF = 13
B2 = 64
B2P = BPW // B2


def _sc_body(xT_hbm, w2_hbm, bias_hbm, emb_hbm, out_hbm, *scratch):
    (idx_v, gidx_v, par_v, whi0, whi1, wr0, wr1, S_v, sqA_v, ffm_v,
     ret_v, bias_v) = scratch[:12]
    row_bufs = scratch[12:12 + HALF]
    sem = scratch[12 + HALF]

    cid = lax.axis_index("c")
    sid = lax.axis_index("s")
    wid = sid * NC + cid
    base = wid * BPW

    pltpu.sync_copy(xT_hbm.at[:, pl.ds(base, BPW)], idx_v)
    pltpu.sync_copy(bias_hbm, bias_v)

    VBH = 25088 // 2
    for f in range(F):
        for g in range(BPW // L):
            sl = pl.ds(g * L, L)
            w = idx_v[f, sl]
            jb = lax.shift_right_logical(
                lax.shift_right_logical(w, 9) * 1338, 16)
            r = w - jb * 25088
            hi = r >= VBH
            gidx_v[f, sl] = jb * VBH + jnp.where(hi, r - VBH, r)
            par_v[f, sl] = jnp.where(hi, L, 0)

    lane = lax.broadcasted_iota(jnp.int32, (L,), 0)
    _dnums = lax.GatherDimensionNumbers(
        offset_dims=(), collapsed_slice_dims=(0,), start_index_map=(0,))

    def _shuffle(v, perm):
        return lax.gather(
            v, perm[:, None], dimension_numbers=_dnums, slice_sizes=(1,),
            mode=lax.GatherScatterMode.PROMISE_IN_BOUNDS)

    def _lane_allsum(v):
        for sh in (1, 2, 4, 8):
            v = v + _shuffle(v, lane ^ sh)
        return v

    for bp in range(B2P):
        bb = bp * B2
        for half in range(2):
            f0 = half * HALF
            descs = [
                pltpu.async_copy(
                    emb_hbm.at[(f0 + j) // 2].at[gidx_v.at[f0 + j, pl.ds(bb, B2)]],
                    row_bufs[j], sem)
                for j in range(HALF)
            ]
            for d_ in descs:
                d_.wait()

            def _chunks(j, b):
                off = ((f0 + j) & 1) * D
                band = b & (L - 1)
                pv = par_v[f0 + j, pl.ds(bb + (b - band), L)]
                sh = _shuffle(pv, jnp.full((L,), band, jnp.int32))
                sh = sh.astype(jnp.uint32)
                regs = []
                for c in range(NCH):
                    u = plsc.bitcast(row_bufs[j][b, pl.ds(off + c * L, L)],
                                     jnp.uint32)
                    bits = lax.shift_left(lax.shift_right_logical(u, sh),
                                          jnp.uint32(16))
                    regs.append(plsc.bitcast(bits, jnp.float32))
                return regs

            if half == 0:
                def body_a(b, carry):
                    v = _chunks(0, b)
                    S = list(v)
                    q = [vv * vv for vv in v]
                    for j in range(1, HALF):
                        v = _chunks(j, b)
                        for c in range(NCH):
                            S[c] = S[c] + v[c]
                            q[c] = q[c] + v[c] * v[c]
                    for c in range(NCH):
                        S_v[b, pl.ds(c * L, L)] = S[c]
                    sqA_v[b, :] = (q[0] + q[1]) + (q[2] + q[3])
                    return carry

                lax.fori_loop(0, B2, body_a, 0, unroll=2)
            else:
                def body_b(b, vec):
                    S = [S_v[b, pl.ds(c * L, L)] for c in range(NCH)]
                    v = _chunks(0, b)
                    q = [vv * vv for vv in v]
                    for c in range(NCH):
                        S[c] = S[c] + v[c]
                    for j in range(1, HALF):
                        v = _chunks(j, b)
                        for c in range(NCH):
                            S[c] = S[c] + v[c]
                            q[c] = q[c] + v[c] * v[c]
                    p0 = S[0] * S[0] - q[0]
                    p1 = S[1] * S[1] - q[1]
                    p2 = S[2] * S[2] - q[2]
                    p3 = S[3] * S[3] - q[3]
                    s = _lane_allsum(((p0 + p1) + (p2 + p3)) - sqA_v[b, :])
                    bi = b & (L - 1)
                    vec = jnp.where(lane == bi, s, vec)

                    @pl.when(bi == L - 1)
                    def _():
                        ffm_v[pl.ds(bb + b - (L - 1), L)] = vec

                    return vec

                lax.fori_loop(0, B2, body_b, jnp.zeros((L,), jnp.float32),
                              unroll=2)

    lane16 = lane
    bias_vec = bias_v[...]
    for g in range(BPW // L):
        sl = pl.ds(g * L, L)
        ret_v[sl] = bias_vec + 0.5 * ffm_v[sl]

    def fire_w(f, whi, wr):
        for g in range(BPW // L):
            sl = pl.ds(g * L, L)
            whi[sl] = lax.shift_right_logical(idx_v[f, sl], 3)
        return pltpu.async_copy(w2_hbm.at[f].at[whi], wr, sem)

    def reduce_w(f, wr):
        for g in range(BPW // L):
            sl = pl.ds(g * L, L)
            rows = lane16 + (g * L)
            cols = idx_v[f, sl] & 7
            vals = plsc.load_gather(wr, [rows, cols])
            plsc.addupdate(ret_v.at[sl], vals)

    slots = [(whi0, wr0), (whi1, wr1)]
    desc = fire_w(0, *slots[0])
    for f in range(F):
        nxt = None
        if f + 1 < F:
            nxt = fire_w(f + 1, *slots[(f + 1) % 2])
        desc.wait()
        reduce_w(f, slots[f % 2][1])
        desc = nxt

    pltpu.sync_copy(ret_v, out_hbm.at[pl.ds(base, BPW)])


_SC_SCRATCH = [
    pltpu.VMEM((F, BPW), jnp.int32),
    pltpu.VMEM((F, BPW), jnp.int32),
    pltpu.VMEM((F, BPW), jnp.int32),
    pltpu.VMEM((BPW,), jnp.int32),
    pltpu.VMEM((BPW,), jnp.int32),
    pltpu.VMEM((BPW, 8), jnp.float32),
    pltpu.VMEM((BPW, 8), jnp.float32),
    pltpu.VMEM((B2, D), jnp.float32),
    pltpu.VMEM((B2, L), jnp.float32),
    pltpu.VMEM((BPW,), jnp.float32),
    pltpu.VMEM((BPW,), jnp.float32),
    pltpu.VMEM((L,), jnp.float32),
] + [pltpu.VMEM((B2, 2 * D), jnp.float32) for _ in range(HALF)] + [
    pltpu.SemaphoreType.DMA,
]

_sc_ffm = functools.partial(
    pl.kernel,
    out_type=jax.ShapeDtypeStruct((B,), jnp.float32),
    mesh=plsc.VectorSubcoreMesh(core_axis_name="c", subcore_axis_name="s"),
    scratch_types=_SC_SCRATCH,
    compiler_params=pltpu.CompilerParams(
        needs_layout_passes=False, use_tc_tiling_on_sc=False),
)(_sc_body)


_VB = 25088
_NJ = -(-V // _VB)
_NWB = 10
_WROWS = 20320
_WPAD = _WROWS * 128 - F * V


def _tc_body(emb_ref, w_ref, bias_ref, out_ref, reg_ref, acc_ref):
    i = pl.program_id(0)
    j = pl.program_id(1)
    k = i * _NJ + j
    xx = emb_ref[...].reshape(2 * D, _VB)
    a = xx[:, : _VB // 2]
    b = xx[:, _VB // 2 :]
    au = lax.bitcast_convert_type(a.astype(jnp.bfloat16), jnp.uint16).astype(jnp.uint32)
    bu = lax.bitcast_convert_type(b.astype(jnp.bfloat16), jnp.uint16).astype(jnp.uint32)
    w32 = lax.bitcast_convert_type(au | (bu << 16), jnp.float32)
    out_ref[0] = jnp.transpose(w32)

    @pl.when(k == 0)
    def _():
        acc_ref[0] = bias_ref[0] * bias_ref[0]

    r0 = lax.broadcasted_iota(jnp.int32, (2 * D, 2 * D), 0)
    r1 = lax.broadcasted_iota(jnp.int32, (2 * D, 2 * D), 1)
    eye = jnp.where(r0 == r1, 1.0, 0.0)
    dn = (((1,), (1,)), ((), ()))

    def _acc(xm):
        g = lax.dot_general(xm, xm, dn, preferred_element_type=jnp.float32)
        acc_ref[0] += jnp.sum(g * eye) * (1.0 / (V * D))

    @pl.when(j < _NJ - 1)
    def _():
        _acc(xx)

    @pl.when(j == _NJ - 1)
    def _():
        vpos = j * _VB + lax.broadcasted_iota(jnp.int32, (2 * D, _VB), 1)
        _acc(jnp.where(vpos < V, xx, 0.0))

    @pl.when(k < _NWB)
    def _():
        w = w_ref[...]
        acc_ref[0] += jnp.sum(w * w) * (1.0 / (F * V))

    @pl.when((i == F // 2 - 1) & (j == _NJ - 1))
    def _():
        reg_ref[0, 0] = acc_ref[0]


def _tc_pack_reg(emb3t, wpad, bias):
    return pl.pallas_call(
        _tc_body,
        grid=(F // 2, _NJ),
        in_specs=[
            pl.BlockSpec((2, D, _VB), lambda i, j: (i, 0, j)),
            pl.BlockSpec((_WROWS // _NWB, 128),
                         lambda i, j: (jnp.minimum(i * _NJ + j, _NWB - 1), 0)),
            pl.BlockSpec(memory_space=pltpu.SMEM),
        ],
        out_specs=[
            pl.BlockSpec((1, _VB // 2, 128), lambda i, j: (i, j, 0)),
            pl.BlockSpec(memory_space=pltpu.SMEM),
        ],
        out_shape=[
            jax.ShapeDtypeStruct((F // 2, _NJ * _VB // 2, 128), jnp.float32),
            jax.ShapeDtypeStruct((1, 1), jnp.float32),
        ],
        scratch_shapes=[pltpu.SMEM((1,), jnp.float32)],
    )(emb3t, wpad, bias)


def kernel(x, W_lin, bias, emb):
    xT = x.T
    w2 = W_lin.reshape(F, V // 8, 8)
    bias16 = jnp.broadcast_to(bias, (L,))

    emb3t = jnp.transpose(emb, (0, 2, 1))
    wflat = W_lin.reshape(F * V)
    wpad = jnp.pad(wflat, (0, _WPAD)).reshape(_WROWS, 128)
    emb13, reg = _tc_pack_reg(emb3t, wpad, bias)

    ret_val = _sc_ffm(xT, w2, bias16, emb13)
    return (ret_val, reg[0, 0])
